# baseline (speedup 1.0000x reference)
import functools

import jax
import jax.numpy as jnp
import numpy as np
from jax import lax
from jax.experimental import pallas as pl
from jax.experimental.pallas import tpu as pltpu
from jax.experimental.pallas import tpu_sc as plsc

B = 4096
F = 26
V = 100000
E = 32
D = 128
NUM_TOKENS = 8
SEED = 0

_perm = np.random.RandomState(SEED).permutation(F)
_GROUPS = [
    _perm[F * i // NUM_TOKENS: F * (i + 1) // NUM_TOKENS].tolist()
    for i in range(NUM_TOKENS)
]

_G = np.zeros((F * E, NUM_TOKENS * E), dtype=np.float32)
_GS = np.zeros((F, NUM_TOKENS), dtype=np.float32)
for _t, _g in enumerate(_GROUPS):
    for _f in _g:
        _GS[_f, _t] = 1.0 / len(_g)
        for _e in range(E):
            _G[_f * E + _e, _t * E + _e] = 1.0 / len(_g)

_NC, _NS = 2, 16
_NW = _NC * _NS
_BF = B * F
_ROWS_PER_W = _BF // _NW
_IDX_CHUNK = 128
_CHUNKS_PER_W = _ROWS_PER_W // _IDX_CHUNK


def _sc_gather_body(idx_hbm, table_hbm, out_hbm, idx_v, rows_v, sem):
    wid = lax.axis_index("s") * _NC + lax.axis_index("c")
    base = wid * _CHUNKS_PER_W
    pltpu.sync_copy(idx_hbm.at[pl.ds(base, _CHUNKS_PER_W)], idx_v)
    copies = []
    for j in range(_CHUNKS_PER_W):
        copies.append(
            pltpu.async_copy(
                table_hbm.at[idx_v.at[j]],
                rows_v.at[pl.ds(j * _IDX_CHUNK, _IDX_CHUNK)],
                sem,
            )
        )
    for c in copies:
        c.wait()
    pltpu.sync_copy(rows_v, out_hbm.at[pl.ds(wid * _ROWS_PER_W, _ROWS_PER_W)])


@functools.cache
def _sc_gather():
    return pl.kernel(
        _sc_gather_body,
        mesh=plsc.VectorSubcoreMesh(
            core_axis_name="c", subcore_axis_name="s", num_cores=_NC
        ),
        out_type=jax.ShapeDtypeStruct((_BF, E), jnp.float32),
        compiler_params=pltpu.CompilerParams(use_tc_tiling_on_sc=False),
        scratch_types=[
            pltpu.VMEM((_CHUNKS_PER_W, _IDX_CHUNK), jnp.int32),
            pltpu.VMEM((_ROWS_PER_W, E), jnp.float32),
            pltpu.SemaphoreType.DMA,
        ],
    )


def _tc_body(feats_ref, mask_ref, g_ref, p_ref, w_ref, b_ref, gamma_ref,
             beta_ref, out_ref):
    x = jnp.dot(feats_ref[...], g_ref[...], preferred_element_type=jnp.float32)
    x = x + jnp.dot(mask_ref[...], p_ref[...],
                    preferred_element_type=jnp.float32)
    gamma = gamma_ref[...]
    beta = beta_ref[...]
    bias = b_ref[...]
    for t in range(NUM_TOKENS):
        xt = x[:, t * E:(t + 1) * E]
        y = jnp.dot(xt, w_ref[...], preferred_element_type=jnp.float32) + bias
        y = y * jax.nn.sigmoid(y)
        mu = jnp.mean(y, axis=-1, keepdims=True)
        var = jnp.mean((y - mu) ** 2, axis=-1, keepdims=True)
        out_ref[:, t, :] = (y - mu) * lax.rsqrt(var + 1e-5) * gamma + beta


_TC_BLOCK = 512


def kernel(int_feats, missing_mask, tables, missing_emb, W, b, gamma, beta):
    offsets = (jnp.arange(F, dtype=jnp.int32) * V)[None, :]
    idx = (int_feats + offsets).reshape(_BF // _IDX_CHUNK, _IDX_CHUNK)
    table = tables[0]

    feats_flat = _sc_gather()(idx, table)
    feats2d = feats_flat.reshape(B, F * E)

    p_mat = (jnp.asarray(_GS)[:, :, None] * missing_emb[0][:, None, :]
             ).reshape(F, NUM_TOKENS * E)

    grid = B // _TC_BLOCK
    out = pl.pallas_call(
        _tc_body,
        grid=(grid,),
        in_specs=[
            pl.BlockSpec((_TC_BLOCK, F * E), lambda i: (i, 0)),
            pl.BlockSpec((_TC_BLOCK, F), lambda i: (i, 0)),
            pl.BlockSpec((F * E, NUM_TOKENS * E), lambda i: (0, 0)),
            pl.BlockSpec((F, NUM_TOKENS * E), lambda i: (0, 0)),
            pl.BlockSpec((E, D), lambda i: (0, 0)),
            pl.BlockSpec((1, D), lambda i: (0, 0)),
            pl.BlockSpec((1, D), lambda i: (0, 0)),
            pl.BlockSpec((1, D), lambda i: (0, 0)),
        ],
        out_specs=pl.BlockSpec((_TC_BLOCK, NUM_TOKENS, D), lambda i: (i, 0, 0)),
        out_shape=jax.ShapeDtypeStruct((B, NUM_TOKENS, D), jnp.float32),
    )(feats2d, missing_mask, jnp.asarray(_G), p_mat, W,
      b.reshape(1, D), gamma.reshape(1, D), beta.reshape(1, D))
    return out
